# Initial kernel scaffold; baseline (speedup 1.0000x reference)
import jax
import jax.numpy as jnp
from jax.experimental import pallas as pl


def kernel(pocket_x, pocket_edge_index, ligand_x, ligand_edge_index, pW1, pb1, pW2, pb2, lW1, lb1, lW2, lb2, hW1, hb1, hW2, hb2):
    raise NotImplementedError("write your pallas kernel here")


# speedup vs baseline: 5.7800x; 5.7800x over previous
import jax
import jax.numpy as jnp
from jax import lax
from jax.experimental import pallas as pl
from jax.experimental.pallas import tpu as pltpu
from jax.experimental.pallas import tpu_sc as plsc

N = 10000
HID = 64
WID = 128
NACC = 10240
DUMMY = N
NC = 2
NS = 16
NW = NC * NS
CHUNK = 128
RPT = NACC // NS

EP = 320000
EL = 160000
PT_P = ((EP // NW) + CHUNK - 1) // CHUNK * CHUNK
PT_L = ((EL // NW) + CHUNK - 1) // CHUNK * CHUNK
PEP = PT_P * NW
PEL = PT_L * NW
NCH_P = PT_P // CHUNK
NCH_L = PT_L // CHUNK

_SC_MESH = plsc.VectorSubcoreMesh(core_axis_name="c", subcore_axis_name="s")


def _rows_body(nch, pt, g_hbm, src_hbm, dst_hbm, s_out, idxs_v, idxd_v,
               rows_v, acc, sem):
    cid = lax.axis_index("c")
    sid = lax.axis_index("s")
    wid = sid * NC + cid

    def fillz(i, _):
        r = i // 8
        c = (i % 8) * 16
        rows_v[r, pl.ds(c, 16)] = jnp.zeros((16,), jnp.float32)
        return 0
    lax.fori_loop(0, CHUNK * 8, fillz, 0)

    row0 = sid * RPT
    for k in range(RPT // CHUNK):
        pltpu.sync_copy(rows_v, acc.at[pl.ds(row0 + k * CHUNK, CHUNK)])
    plsc.subcore_barrier()

    def step(j, _):
        base = wid * pt + j * CHUNK
        pltpu.sync_copy(src_hbm.at[pl.ds(base, CHUNK)], idxs_v)
        pltpu.sync_copy(dst_hbm.at[pl.ds(base, CHUNK)], idxd_v.at[0])
        pltpu.async_copy(g_hbm.at[idxs_v], rows_v, sem).wait()
        pltpu.sync_copy(rows_v, acc.at[idxd_v.at[0]], add=True)
        return 0
    lax.fori_loop(0, nch, step, 0)

    plsc.subcore_barrier()
    pltpu.sync_copy(acc.at[pl.ds(row0, RPT)],
                    s_out.at[cid, pl.ds(row0, RPT)])


@jax.jit
def _sc_rows_p(g, src, dst):
    def body(*a):
        return _rows_body(NCH_P, PT_P, *a)
    f = pl.kernel(
        body,
        out_type=jax.ShapeDtypeStruct((NC, NACC, WID), jnp.float32),
        mesh=_SC_MESH,
        scratch_types=[
            pltpu.VMEM((CHUNK,), jnp.int32),
            pltpu.VMEM((1, CHUNK), jnp.int32),
            pltpu.VMEM((CHUNK, WID), jnp.float32),
            pltpu.VMEM_SHARED((NACC, WID), jnp.float32),
            pltpu.SemaphoreType.DMA,
        ],
    )
    return f(g, src, dst)


@jax.jit
def _sc_rows_l(g, src, dst):
    def body(*a):
        return _rows_body(NCH_L, PT_L, *a)
    f = pl.kernel(
        body,
        out_type=jax.ShapeDtypeStruct((NC, NACC, WID), jnp.float32),
        mesh=_SC_MESH,
        scratch_types=[
            pltpu.VMEM((CHUNK,), jnp.int32),
            pltpu.VMEM((1, CHUNK), jnp.int32),
            pltpu.VMEM((CHUNK, WID), jnp.float32),
            pltpu.VMEM_SHARED((NACC, WID), jnp.float32),
            pltpu.SemaphoreType.DMA,
        ],
    )
    return f(g, src, dst)


BR = 2000


def _dinv_of(degp_blk):
    deg = degp_blk[0, :, 0:1] + degp_blk[1, :, 0:1] + 1.0
    return lax.rsqrt(deg)


def _pad128(v):
    return jnp.concatenate([v, jnp.zeros_like(v)], axis=1)


def _t1_body(x_ref, w_ref, degp_ref, o_ref):
    dinv = _dinv_of(degp_ref[...])
    h = jnp.dot(x_ref[...], w_ref[...], preferred_element_type=jnp.float32)
    o_ref[...] = _pad128(h * dinv)


@jax.jit
def _t1(x, w, degp):
    din = x.shape[1]
    return pl.pallas_call(
        _t1_body,
        grid=(N // BR,),
        in_specs=[
            pl.BlockSpec((BR, din), lambda i: (i, 0)),
            pl.BlockSpec((din, HID), lambda i: (0, 0)),
            pl.BlockSpec((NC, BR, WID), lambda i: (0, i, 0)),
        ],
        out_specs=pl.BlockSpec((BR, WID), lambda i: (i, 0)),
        out_shape=jax.ShapeDtypeStruct((N, WID), jnp.float32),
    )(x, w, degp)


def _layer_z(s_ref, g_ref, degp_ref, b_ref):
    dinv = _dinv_of(degp_ref[...])
    s = s_ref[0, :, 0:HID] + s_ref[1, :, 0:HID] + g_ref[:, 0:HID]
    return jnp.maximum(s * dinv + b_ref[...], 0.0)


def _t2_body(s_ref, g_ref, degp_ref, b_ref, w_ref, o_ref):
    dinv = _dinv_of(degp_ref[...])
    z = _layer_z(s_ref, g_ref[...], degp_ref, b_ref)
    h = jnp.dot(z, w_ref[...], preferred_element_type=jnp.float32)
    o_ref[...] = _pad128(h * dinv)


@jax.jit
def _t2(s, g, degp, b, w):
    return pl.pallas_call(
        _t2_body,
        grid=(N // BR,),
        in_specs=[
            pl.BlockSpec((NC, BR, WID), lambda i: (0, i, 0)),
            pl.BlockSpec((BR, WID), lambda i: (i, 0)),
            pl.BlockSpec((NC, BR, WID), lambda i: (0, i, 0)),
            pl.BlockSpec((1, HID), lambda i: (0, 0)),
            pl.BlockSpec((HID, HID), lambda i: (0, 0)),
        ],
        out_specs=pl.BlockSpec((BR, WID), lambda i: (i, 0)),
        out_shape=jax.ShapeDtypeStruct((N, WID), jnp.float32),
    )(s, g, degp, b, w)


def _t3_body(s_ref, g_ref, degp_ref, b_ref, o_ref):
    z = _layer_z(s_ref, g_ref[...], degp_ref, b_ref)
    part = jnp.sum(z, axis=0, keepdims=True)

    @pl.when(pl.program_id(0) == 0)
    def _():
        o_ref[...] = jnp.zeros_like(o_ref)

    o_ref[...] += part


@jax.jit
def _t3(s, g, degp, b):
    return pl.pallas_call(
        _t3_body,
        grid=(N // BR,),
        in_specs=[
            pl.BlockSpec((NC, BR, WID), lambda i: (0, i, 0)),
            pl.BlockSpec((BR, WID), lambda i: (i, 0)),
            pl.BlockSpec((NC, BR, WID), lambda i: (0, i, 0)),
            pl.BlockSpec((1, HID), lambda i: (0, 0)),
        ],
        out_specs=pl.BlockSpec((1, HID), lambda i: (0, 0)),
        out_shape=jax.ShapeDtypeStruct((1, HID), jnp.float32),
    )(s, g, degp, b)


def _t4_body(ps_ref, ls_ref, w1_ref, b1_ref, w2t_ref, b2_ref, o_ref):
    cat = jnp.concatenate([ps_ref[...], ls_ref[...]], axis=1) * (1.0 / N)
    h = jnp.maximum(
        jnp.dot(cat, w1_ref[...], preferred_element_type=jnp.float32)
        + b1_ref[...], 0.0)
    o_ref[...] = jnp.sum(h * w2t_ref[...], axis=1, keepdims=True) + b2_ref[...]


@jax.jit
def _t4(ps, ls, w1, b1, w2t, b2):
    return pl.pallas_call(
        _t4_body,
        out_shape=jax.ShapeDtypeStruct((1, 1), jnp.float32),
    )(ps, ls, w1, b1, w2t, b2)


@jax.jit
def kernel(pocket_x, pocket_edge_index, ligand_x, ligand_edge_index,
           pW1, pb1, pW2, pb2, lW1, lb1, lW2, lb2, hW1, hb1, hW2, hb2):
    srcp = pocket_edge_index[0].astype(jnp.int32)
    dstp = pocket_edge_index[1].astype(jnp.int32)
    srcl = ligand_edge_index[0].astype(jnp.int32)
    dstl = ligand_edge_index[1].astype(jnp.int32)

    srcp = jnp.concatenate([srcp, jnp.zeros((PEP - EP,), jnp.int32)])
    dstp = jnp.concatenate([dstp, jnp.full((PEP - EP,), DUMMY, jnp.int32)])
    srcl = jnp.concatenate([srcl, jnp.zeros((PEL - EL,), jnp.int32)])
    dstl = jnp.concatenate([dstl, jnp.full((PEL - EL,), DUMMY, jnp.int32)])

    ones_tab = jnp.ones((N, WID), jnp.float32)
    degp = _sc_rows_p(ones_tab, srcp, dstp)
    degl = _sc_rows_l(ones_tab, srcl, dstl)

    g1p = _t1(pocket_x, pW1, degp)
    g1l = _t1(ligand_x, lW1, degl)

    s1p = _sc_rows_p(g1p, srcp, dstp)
    s1l = _sc_rows_l(g1l, srcl, dstl)

    g2p = _t2(s1p, g1p, degp, pb1.reshape(1, HID), pW2)
    g2l = _t2(s1l, g1l, degl, lb1.reshape(1, HID), lW2)

    s2p = _sc_rows_p(g2p, srcp, dstp)
    s2l = _sc_rows_l(g2l, srcl, dstl)

    psum = _t3(s2p, g2p, degp, pb2.reshape(1, HID))
    lsum = _t3(s2l, g2l, degl, lb2.reshape(1, HID))

    out = _t4(psum, lsum, hW1, hb1.reshape(1, HID),
              hW2.reshape(1, HID), hb2.reshape(1, 1))
    return out.reshape((1,))


# speedup vs baseline: 7.1328x; 1.2341x over previous
import jax
import jax.numpy as jnp
from jax import lax
from jax.experimental import pallas as pl
from jax.experimental.pallas import tpu as pltpu
from jax.experimental.pallas import tpu_sc as plsc

N = 10000
HID = 64
WID = 128
NACC = 10240
DUMMY = N
NC = 2
NS = 16
NW = NC * NS
CHUNK = 128
NBUF = 2
RPT = NACC // NS

EP = 320000
EL = 160000
PT_P = ((EP // NW) + 4 * CHUNK - 1) // (4 * CHUNK) * (4 * CHUNK)
PT_L = ((EL // NW) + 4 * CHUNK - 1) // (4 * CHUNK) * (4 * CHUNK)
PEP = PT_P * NW
PEL = PT_L * NW
NCH_P = PT_P // CHUNK
NCH_L = PT_L // CHUNK

_SC_MESH = plsc.VectorSubcoreMesh(core_axis_name="c", subcore_axis_name="s")


def _zero_fill(buf, w):
    def fillz(i, _):
        r = i // (w // 16)
        c = (i % (w // 16)) * 16
        buf[r, pl.ds(c, 16)] = jnp.zeros((16,), jnp.float32)
        return 0
    lax.fori_loop(0, CHUNK * (w // 16), fillz, 0)


def _rows_body(nch, nphase, g_hbm, src2_hbm, dst2_hbm, s_out, srcb, dstb,
               rows, acc, *sems):
    gsem = sems[:NBUF]
    ssem = sems[NBUF:]
    cid = lax.axis_index("c")
    sid = lax.axis_index("s")
    wid = sid * NC + cid
    row0 = sid * RPT
    half = nch // nphase

    _zero_fill(rows.at[0], WID)
    for k in range(RPT // CHUNK):
        pltpu.sync_copy(rows.at[0], acc.at[pl.ds(row0 + k * CHUNK, CHUNK)])
    plsc.subcore_barrier()

    for phase in range(nphase):
        base_ch = wid * nch + phase * half
        pltpu.sync_copy(src2_hbm.at[pl.ds(base_ch, half)], srcb)
        pltpu.sync_copy(dst2_hbm.at[pl.ds(base_ch, half)], dstb)

        for b in range(NBUF):
            pltpu.async_copy(g_hbm.at[srcb.at[b]], rows.at[b], gsem[b])

        def step(k, _):
            for b in range(NBUF):
                c = NBUF * k + b
                pltpu.make_async_copy(g_hbm.at[srcb.at[c]], rows.at[b],
                                      gsem[b]).wait()
                pltpu.async_copy(rows.at[b], acc.at[dstb.at[c]], ssem[b],
                                 add=True)

                @pl.when(k < half // NBUF - 1)
                def _():
                    pltpu.make_async_copy(rows.at[b], acc.at[dstb.at[c]],
                                          ssem[b]).wait()
                    pltpu.async_copy(g_hbm.at[srcb.at[c + NBUF]], rows.at[b],
                                     gsem[b])
            return 0
        lax.fori_loop(0, half // NBUF, step, 0)

        for b in range(NBUF):
            c = half - NBUF + b
            pltpu.make_async_copy(rows.at[b], acc.at[dstb.at[c]],
                                  ssem[b]).wait()

    plsc.subcore_barrier()
    pltpu.sync_copy(acc.at[pl.ds(row0, RPT)],
                    s_out.at[cid, pl.ds(row0, RPT)])


def _deg_body(dstp2_hbm, dstl2_hbm, dp_out, dl_out, dstpb, dstlb, ones_v,
              acc, *ssem):
    nsem = len(ssem)
    cid = lax.axis_index("c")
    sid = lax.axis_index("s")
    wid = sid * NC + cid
    row0 = sid * RPT

    def fillo(i, _):
        c = (i % (WID // 16)) * 16
        ones_v[i // (WID // 16), pl.ds(c, 16)] = jnp.full((16,), 1.0,
                                                          jnp.float32)
        return 0

    pltpu.sync_copy(dstp2_hbm.at[pl.ds(wid * NCH_P, NCH_P)], dstpb)
    pltpu.sync_copy(dstl2_hbm.at[pl.ds(wid * NCH_L, NCH_L)], dstlb)

    for out, dstb, nch in ((dp_out, dstpb, NCH_P), (dl_out, dstlb, NCH_L)):
        _zero_fill(ones_v, WID)
        for k in range(RPT // CHUNK):
            pltpu.sync_copy(ones_v, acc.at[pl.ds(row0 + k * CHUNK, CHUNK)])
        lax.fori_loop(0, CHUNK * (WID // 16), fillo, 0)
        plsc.subcore_barrier()

        for b in range(nsem):
            pltpu.async_copy(ones_v, acc.at[dstb.at[b]], ssem[b], add=True)

        def step(k, _):
            for b in range(nsem):
                c = nsem * k + b
                pltpu.make_async_copy(ones_v, acc.at[dstb.at[c]],
                                      ssem[b]).wait()

                @pl.when(k < nch // nsem - 1)
                def _():
                    pltpu.async_copy(ones_v, acc.at[dstb.at[c + nsem]],
                                     ssem[b], add=True)
            return 0
        lax.fori_loop(0, nch // nsem, step, 0)

        plsc.subcore_barrier()
        pltpu.sync_copy(acc.at[pl.ds(row0, RPT)],
                        out.at[cid, pl.ds(row0, RPT)])


def _make_rows(nch, nphase):
    def body(*a):
        return _rows_body(nch, nphase, *a)
    return pl.kernel(
        body,
        out_type=jax.ShapeDtypeStruct((NC, NACC, WID), jnp.float32),
        mesh=_SC_MESH,
        scratch_types=[
            pltpu.VMEM((nch // nphase, CHUNK), jnp.int32),
            pltpu.VMEM((nch // nphase, CHUNK), jnp.int32),
            pltpu.VMEM((NBUF, CHUNK, WID), jnp.float32),
            pltpu.VMEM_SHARED((NACC, WID), jnp.float32),
        ] + [pltpu.SemaphoreType.DMA] * (2 * NBUF),
    )


def _make_deg():
    return pl.kernel(
        _deg_body,
        out_type=(jax.ShapeDtypeStruct((NC, NACC, WID), jnp.float32),
                  jax.ShapeDtypeStruct((NC, NACC, WID), jnp.float32)),
        mesh=_SC_MESH,
        scratch_types=[
            pltpu.VMEM((NCH_P, CHUNK), jnp.int32),
            pltpu.VMEM((NCH_L, CHUNK), jnp.int32),
            pltpu.VMEM((CHUNK, WID), jnp.float32),
            pltpu.VMEM_SHARED((NACC, WID), jnp.float32),
        ] + [pltpu.SemaphoreType.DMA] * 4,
    )


@jax.jit
def _sc_rows_p(g, src2, dst2):
    return _make_rows(NCH_P, 2)(g, src2, dst2)


@jax.jit
def _sc_rows_l(g, src2, dst2):
    return _make_rows(NCH_L, 1)(g, src2, dst2)


@jax.jit
def _sc_deg(dstp2, dstl2):
    return _make_deg()(dstp2, dstl2)


BR = 2000


def _dinv_of(degp_blk):
    deg = degp_blk[0, :, 0:1] + degp_blk[1, :, 0:1] + 1.0
    return lax.rsqrt(deg)


def _pad128(v):
    return jnp.concatenate([v, jnp.zeros_like(v)], axis=1)


def _t1_body(x_ref, w_ref, degp_ref, o_ref):
    dinv = _dinv_of(degp_ref[...])
    h = jnp.dot(x_ref[...], w_ref[...], preferred_element_type=jnp.float32)
    o_ref[...] = _pad128(h * dinv)


@jax.jit
def _t1(x, w, degp):
    din = x.shape[1]
    return pl.pallas_call(
        _t1_body,
        grid=(N // BR,),
        in_specs=[
            pl.BlockSpec((BR, din), lambda i: (i, 0)),
            pl.BlockSpec((din, HID), lambda i: (0, 0)),
            pl.BlockSpec((NC, BR, WID), lambda i: (0, i, 0)),
        ],
        out_specs=pl.BlockSpec((BR, WID), lambda i: (i, 0)),
        out_shape=jax.ShapeDtypeStruct((N, WID), jnp.float32),
    )(x, w, degp)


def _layer_z(s_ref, g_ref, degp_ref, b_ref):
    dinv = _dinv_of(degp_ref[...])
    s = s_ref[0, :, 0:HID] + s_ref[1, :, 0:HID] + g_ref[:, 0:HID]
    return jnp.maximum(s * dinv + b_ref[...], 0.0)


def _t2_body(s_ref, g_ref, degp_ref, b_ref, w_ref, o_ref):
    dinv = _dinv_of(degp_ref[...])
    z = _layer_z(s_ref, g_ref[...], degp_ref, b_ref)
    h = jnp.dot(z, w_ref[...], preferred_element_type=jnp.float32)
    o_ref[...] = _pad128(h * dinv)


@jax.jit
def _t2(s, g, degp, b, w):
    return pl.pallas_call(
        _t2_body,
        grid=(N // BR,),
        in_specs=[
            pl.BlockSpec((NC, BR, WID), lambda i: (0, i, 0)),
            pl.BlockSpec((BR, WID), lambda i: (i, 0)),
            pl.BlockSpec((NC, BR, WID), lambda i: (0, i, 0)),
            pl.BlockSpec((1, HID), lambda i: (0, 0)),
            pl.BlockSpec((HID, HID), lambda i: (0, 0)),
        ],
        out_specs=pl.BlockSpec((BR, WID), lambda i: (i, 0)),
        out_shape=jax.ShapeDtypeStruct((N, WID), jnp.float32),
    )(s, g, degp, b, w)


def _t3_body(s_ref, g_ref, degp_ref, b_ref, o_ref):
    z = _layer_z(s_ref, g_ref[...], degp_ref, b_ref)
    part = jnp.sum(z, axis=0, keepdims=True)

    @pl.when(pl.program_id(0) == 0)
    def _():
        o_ref[...] = jnp.zeros_like(o_ref)

    o_ref[...] += part


@jax.jit
def _t3(s, g, degp, b):
    return pl.pallas_call(
        _t3_body,
        grid=(N // BR,),
        in_specs=[
            pl.BlockSpec((NC, BR, WID), lambda i: (0, i, 0)),
            pl.BlockSpec((BR, WID), lambda i: (i, 0)),
            pl.BlockSpec((NC, BR, WID), lambda i: (0, i, 0)),
            pl.BlockSpec((1, HID), lambda i: (0, 0)),
        ],
        out_specs=pl.BlockSpec((1, HID), lambda i: (0, 0)),
        out_shape=jax.ShapeDtypeStruct((1, HID), jnp.float32),
    )(s, g, degp, b)


def _t4_body(ps_ref, ls_ref, w1_ref, b1_ref, w2t_ref, b2_ref, o_ref):
    cat = jnp.concatenate([ps_ref[...], ls_ref[...]], axis=1) * (1.0 / N)
    h = jnp.maximum(
        jnp.dot(cat, w1_ref[...], preferred_element_type=jnp.float32)
        + b1_ref[...], 0.0)
    o_ref[...] = jnp.sum(h * w2t_ref[...], axis=1, keepdims=True) + b2_ref[...]


@jax.jit
def _t4(ps, ls, w1, b1, w2t, b2):
    return pl.pallas_call(
        _t4_body,
        out_shape=jax.ShapeDtypeStruct((1, 1), jnp.float32),
    )(ps, ls, w1, b1, w2t, b2)


def _pad2d(v, pe, e, fill):
    return jnp.concatenate(
        [v, jnp.full((pe - e,), fill, jnp.int32)]).reshape(pe // CHUNK, CHUNK)


@jax.jit
def kernel(pocket_x, pocket_edge_index, ligand_x, ligand_edge_index,
           pW1, pb1, pW2, pb2, lW1, lb1, lW2, lb2, hW1, hb1, hW2, hb2):
    srcp = _pad2d(pocket_edge_index[0].astype(jnp.int32), PEP, EP, 0)
    dstp = _pad2d(pocket_edge_index[1].astype(jnp.int32), PEP, EP, DUMMY)
    srcl = _pad2d(ligand_edge_index[0].astype(jnp.int32), PEL, EL, 0)
    dstl = _pad2d(ligand_edge_index[1].astype(jnp.int32), PEL, EL, DUMMY)

    degp, degl = _sc_deg(dstp, dstl)

    g1p = _t1(pocket_x, pW1, degp)
    g1l = _t1(ligand_x, lW1, degl)

    s1p = _sc_rows_p(g1p, srcp, dstp)
    s1l = _sc_rows_l(g1l, srcl, dstl)

    g2p = _t2(s1p, g1p, degp, pb1.reshape(1, HID), pW2)
    g2l = _t2(s1l, g1l, degl, lb1.reshape(1, HID), lW2)

    s2p = _sc_rows_p(g2p, srcp, dstp)
    s2l = _sc_rows_l(g2l, srcl, dstl)

    psum = _t3(s2p, g2p, degp, pb2.reshape(1, HID))
    lsum = _t3(s2l, g2l, degl, lb2.reshape(1, HID))

    out = _t4(psum, lsum, hW1, hb1.reshape(1, HID),
              hW2.reshape(1, HID), hb2.reshape(1, 1))
    return out.reshape((1,))


# speedup vs baseline: 7.2677x; 1.0189x over previous
import jax
import jax.numpy as jnp
from jax import lax
from jax.experimental import pallas as pl
from jax.experimental.pallas import tpu as pltpu
from jax.experimental.pallas import tpu_sc as plsc

N = 10000
HID = 64
WID = 128
NACC = 10240
DUMMY = N
NC = 2
NS = 16
NW = NC * NS
CHUNK = 128
NBUF = 2
RPT = NACC // NS

EP = 320000
EL = 160000
PT_P = ((EP // NW) + 4 * CHUNK - 1) // (4 * CHUNK) * (4 * CHUNK)
PT_L = ((EL // NW) + 4 * CHUNK - 1) // (4 * CHUNK) * (4 * CHUNK)
PEP = PT_P * NW
PEL = PT_L * NW
NCH_P = PT_P // CHUNK
NCH_L = PT_L // CHUNK

_SC_MESH = plsc.VectorSubcoreMesh(core_axis_name="c", subcore_axis_name="s")


def _zero_fill(buf, w):
    def fillz(i, _):
        r = i // (w // 16)
        c = (i % (w // 16)) * 16
        buf[r, pl.ds(c, 16)] = jnp.zeros((16,), jnp.float32)
        return 0
    lax.fori_loop(0, CHUNK * (w // 16), fillz, 0)


def _rows_body(nch, nphase, g_hbm, src2_hbm, dst2_hbm, s_out, srcb, dstb,
               rows, acc, *sems):
    gsem = sems[:NBUF]
    ssem = sems[NBUF:]
    cid = lax.axis_index("c")
    sid = lax.axis_index("s")
    wid = sid * NC + cid
    row0 = sid * RPT
    half = nch // nphase

    _zero_fill(rows.at[0], WID)
    for k in range(RPT // CHUNK):
        pltpu.sync_copy(rows.at[0], acc.at[pl.ds(row0 + k * CHUNK, CHUNK)])
    plsc.subcore_barrier()

    for phase in range(nphase):
        base_ch = wid * nch + phase * half
        pltpu.sync_copy(src2_hbm.at[pl.ds(base_ch, half)], srcb)
        pltpu.sync_copy(dst2_hbm.at[pl.ds(base_ch, half)], dstb)

        for b in range(NBUF):
            pltpu.async_copy(g_hbm.at[srcb.at[b]], rows.at[b], gsem[b])

        def step(k, _):
            for b in range(NBUF):
                c = NBUF * k + b
                pltpu.make_async_copy(g_hbm.at[srcb.at[c]], rows.at[b],
                                      gsem[b]).wait()
                pltpu.async_copy(rows.at[b], acc.at[dstb.at[c]], ssem[b],
                                 add=True)

                @pl.when(k < half // NBUF - 1)
                def _():
                    pltpu.make_async_copy(rows.at[b], acc.at[dstb.at[c]],
                                          ssem[b]).wait()
                    pltpu.async_copy(g_hbm.at[srcb.at[c + NBUF]], rows.at[b],
                                     gsem[b])
            return 0
        lax.fori_loop(0, half // NBUF, step, 0)

        for b in range(NBUF):
            c = half - NBUF + b
            pltpu.make_async_copy(rows.at[b], acc.at[dstb.at[c]],
                                  ssem[b]).wait()

    plsc.subcore_barrier()
    pltpu.sync_copy(acc.at[pl.ds(row0, RPT)],
                    s_out.at[cid, pl.ds(row0, RPT)])


def _deg_body(nch, dst2_hbm, s_out, dstb, ones_v, acc, *ssem):
    nsem = len(ssem)
    cid = lax.axis_index("c")
    sid = lax.axis_index("s")
    wid = sid * NC + cid
    row0 = sid * RPT

    _zero_fill(ones_v, WID)
    for k in range(RPT // CHUNK):
        pltpu.sync_copy(ones_v, acc.at[pl.ds(row0 + k * CHUNK, CHUNK)])

    def fillo(i, _):
        c = (i % (WID // 16)) * 16
        ones_v[i // (WID // 16), pl.ds(c, 16)] = jnp.full((16,), 1.0,
                                                          jnp.float32)
        return 0
    lax.fori_loop(0, CHUNK * (WID // 16), fillo, 0)
    plsc.subcore_barrier()

    pltpu.sync_copy(dst2_hbm.at[pl.ds(wid * nch, nch)], dstb)

    for b in range(nsem):
        pltpu.async_copy(ones_v, acc.at[dstb.at[b]], ssem[b], add=True)

    def step(k, _):
        for b in range(nsem):
            c = nsem * k + b
            pltpu.make_async_copy(ones_v, acc.at[dstb.at[c]], ssem[b]).wait()

            @pl.when(k < nch // nsem - 1)
            def _():
                pltpu.async_copy(ones_v, acc.at[dstb.at[c + nsem]], ssem[b],
                                 add=True)
        return 0
    lax.fori_loop(0, nch // nsem, step, 0)

    plsc.subcore_barrier()
    pltpu.sync_copy(acc.at[pl.ds(row0, RPT)],
                    s_out.at[cid, pl.ds(row0, RPT)])


def _make_rows(nch, nphase):
    def body(*a):
        return _rows_body(nch, nphase, *a)
    return pl.kernel(
        body,
        out_type=jax.ShapeDtypeStruct((NC, NACC, WID), jnp.float32),
        mesh=_SC_MESH,
        scratch_types=[
            pltpu.VMEM((nch // nphase, CHUNK), jnp.int32),
            pltpu.VMEM((nch // nphase, CHUNK), jnp.int32),
            pltpu.VMEM((NBUF, CHUNK, WID), jnp.float32),
            pltpu.VMEM_SHARED((NACC, WID), jnp.float32),
        ] + [pltpu.SemaphoreType.DMA] * (2 * NBUF),
    )


def _make_deg(nch):
    def body(*a):
        return _deg_body(nch, *a)
    return pl.kernel(
        body,
        out_type=jax.ShapeDtypeStruct((NC, NACC, WID), jnp.float32),
        mesh=_SC_MESH,
        scratch_types=[
            pltpu.VMEM((nch, CHUNK), jnp.int32),
            pltpu.VMEM((CHUNK, WID), jnp.float32),
            pltpu.VMEM_SHARED((NACC, WID), jnp.float32),
        ] + [pltpu.SemaphoreType.DMA] * 4,
    )


@jax.jit
def _sc_rows_p(g, src2, dst2):
    return _make_rows(NCH_P, 2)(g, src2, dst2)


@jax.jit
def _sc_rows_l(g, src2, dst2):
    return _make_rows(NCH_L, 1)(g, src2, dst2)


@jax.jit
def _sc_deg_p(dst2):
    return _make_deg(NCH_P)(dst2)


@jax.jit
def _sc_deg_l(dst2):
    return _make_deg(NCH_L)(dst2)


BR = 2000


def _dinv_of(degp_blk):
    deg = degp_blk[0, :, 0:1] + degp_blk[1, :, 0:1] + 1.0
    return lax.rsqrt(deg)


def _pad128(v):
    return jnp.concatenate([v, jnp.zeros_like(v)], axis=1)


def _t1_body(x_ref, w_ref, degp_ref, o_ref):
    dinv = _dinv_of(degp_ref[...])
    h = jnp.dot(x_ref[...], w_ref[...], preferred_element_type=jnp.float32)
    o_ref[...] = _pad128(h * dinv)


@jax.jit
def _t1(x, w, degp):
    din = x.shape[1]
    return pl.pallas_call(
        _t1_body,
        grid=(N // BR,),
        in_specs=[
            pl.BlockSpec((BR, din), lambda i: (i, 0)),
            pl.BlockSpec((din, HID), lambda i: (0, 0)),
            pl.BlockSpec((NC, BR, WID), lambda i: (0, i, 0)),
        ],
        out_specs=pl.BlockSpec((BR, WID), lambda i: (i, 0)),
        out_shape=jax.ShapeDtypeStruct((N, WID), jnp.float32),
    )(x, w, degp)


def _layer_z(s_ref, g_ref, degp_ref, b_ref):
    dinv = _dinv_of(degp_ref[...])
    s = s_ref[0, :, 0:HID] + s_ref[1, :, 0:HID] + g_ref[:, 0:HID]
    return jnp.maximum(s * dinv + b_ref[...], 0.0)


def _t2_body(s_ref, g_ref, degp_ref, b_ref, w_ref, o_ref):
    dinv = _dinv_of(degp_ref[...])
    z = _layer_z(s_ref, g_ref[...], degp_ref, b_ref)
    h = jnp.dot(z, w_ref[...], preferred_element_type=jnp.float32)
    o_ref[...] = _pad128(h * dinv)


@jax.jit
def _t2(s, g, degp, b, w):
    return pl.pallas_call(
        _t2_body,
        grid=(N // BR,),
        in_specs=[
            pl.BlockSpec((NC, BR, WID), lambda i: (0, i, 0)),
            pl.BlockSpec((BR, WID), lambda i: (i, 0)),
            pl.BlockSpec((NC, BR, WID), lambda i: (0, i, 0)),
            pl.BlockSpec((1, HID), lambda i: (0, 0)),
            pl.BlockSpec((HID, HID), lambda i: (0, 0)),
        ],
        out_specs=pl.BlockSpec((BR, WID), lambda i: (i, 0)),
        out_shape=jax.ShapeDtypeStruct((N, WID), jnp.float32),
    )(s, g, degp, b, w)


def _t3_body(s_ref, g_ref, degp_ref, b_ref, o_ref):
    z = _layer_z(s_ref, g_ref[...], degp_ref, b_ref)
    part = jnp.sum(z, axis=0, keepdims=True)

    @pl.when(pl.program_id(0) == 0)
    def _():
        o_ref[...] = jnp.zeros_like(o_ref)

    o_ref[...] += part


@jax.jit
def _t3(s, g, degp, b):
    return pl.pallas_call(
        _t3_body,
        grid=(N // BR,),
        in_specs=[
            pl.BlockSpec((NC, BR, WID), lambda i: (0, i, 0)),
            pl.BlockSpec((BR, WID), lambda i: (i, 0)),
            pl.BlockSpec((NC, BR, WID), lambda i: (0, i, 0)),
            pl.BlockSpec((1, HID), lambda i: (0, 0)),
        ],
        out_specs=pl.BlockSpec((1, HID), lambda i: (0, 0)),
        out_shape=jax.ShapeDtypeStruct((1, HID), jnp.float32),
    )(s, g, degp, b)


def _t4_body(ps_ref, ls_ref, w1_ref, b1_ref, w2t_ref, b2_ref, o_ref):
    cat = jnp.concatenate([ps_ref[...], ls_ref[...]], axis=1) * (1.0 / N)
    h = jnp.maximum(
        jnp.dot(cat, w1_ref[...], preferred_element_type=jnp.float32)
        + b1_ref[...], 0.0)
    o_ref[...] = jnp.sum(h * w2t_ref[...], axis=1, keepdims=True) + b2_ref[...]


@jax.jit
def _t4(ps, ls, w1, b1, w2t, b2):
    return pl.pallas_call(
        _t4_body,
        out_shape=jax.ShapeDtypeStruct((1, 1), jnp.float32),
    )(ps, ls, w1, b1, w2t, b2)


def _pad2d(v, pe, e, fill):
    return jnp.concatenate(
        [v, jnp.full((pe - e,), fill, jnp.int32)]).reshape(pe // CHUNK, CHUNK)


@jax.jit
def kernel(pocket_x, pocket_edge_index, ligand_x, ligand_edge_index,
           pW1, pb1, pW2, pb2, lW1, lb1, lW2, lb2, hW1, hb1, hW2, hb2):
    srcp = _pad2d(pocket_edge_index[0].astype(jnp.int32), PEP, EP, 0)
    dstp = _pad2d(pocket_edge_index[1].astype(jnp.int32), PEP, EP, DUMMY)
    srcl = _pad2d(ligand_edge_index[0].astype(jnp.int32), PEL, EL, 0)
    dstl = _pad2d(ligand_edge_index[1].astype(jnp.int32), PEL, EL, DUMMY)

    degp = _sc_deg_p(dstp)
    degl = _sc_deg_l(dstl)

    g1p = _t1(pocket_x, pW1, degp)
    g1l = _t1(ligand_x, lW1, degl)

    s1p = _sc_rows_p(g1p, srcp, dstp)
    s1l = _sc_rows_l(g1l, srcl, dstl)

    g2p = _t2(s1p, g1p, degp, pb1.reshape(1, HID), pW2)
    g2l = _t2(s1l, g1l, degl, lb1.reshape(1, HID), lW2)

    s2p = _sc_rows_p(g2p, srcp, dstp)
    s2l = _sc_rows_l(g2l, srcl, dstl)

    psum = _t3(s2p, g2p, degp, pb2.reshape(1, HID))
    lsum = _t3(s2l, g2l, degl, lb2.reshape(1, HID))

    out = _t4(psum, lsum, hW1, hb1.reshape(1, HID),
              hW2.reshape(1, HID), hb2.reshape(1, 1))
    return out.reshape((1,))
